# Initial kernel scaffold; baseline (speedup 1.0000x reference)
import jax
import jax.numpy as jnp
from jax.experimental import pallas as pl


def kernel(x, W1, b1, W2, b2):
    raise NotImplementedError("write your pallas kernel here")


# speedup vs baseline: 1594.6169x; 1594.6169x over previous
import jax
import jax.numpy as jnp
from jax.experimental import pallas as pl


_B, _N, _F = 8, 256, 6


def _gnn_kernel(x_ref, w1_ref, b1_ref, w2_ref, b2_ref, out_ref):
    w1 = w1_ref[...]
    b1 = b1_ref[...]
    w2 = w2_ref[...]
    b2 = b2_ref[...]
    for b in range(_B):
        f = x_ref[b]
        absum = jnp.sum(jnp.abs(f), axis=1, keepdims=True)
        a = (absum != 0).astype(jnp.float32)
        c = ((f[:, 3:4] != 0) | (f[:, 4:5] != 0)).astype(jnp.float32)
        nv = jnp.sum(a)
        nc = jnp.sum(a * c)
        deg = a * (nc + c * (nv - nc - 1.0) + 1.0)
        d = jnp.where(deg > 0, 1.0 / jnp.sqrt(jnp.maximum(deg, 1e-12)), 0.0)
        dc = d * c
        dd1c = d * d * (1.0 - c)
        hw = jax.lax.dot_general(
            f, w1, (((1,), (0,)), ((), ())),
            preferred_element_type=jnp.float32)
        S0 = jnp.sum(d * hw, axis=0, keepdims=True)
        S1 = jnp.sum(dc * hw, axis=0, keepdims=True)
        pre = d * S1 + dc * (S0 - S1) + dd1c * hw + b1
        h1 = jnp.maximum(pre, 0.0)
        sd = jnp.sum(d)
        sdc = jnp.sum(dc)
        A0 = jnp.sum(d * h1, axis=0, keepdims=True)
        A1 = jnp.sum(dc * h1, axis=0, keepdims=True)
        A2 = jnp.sum(dd1c * h1, axis=0, keepdims=True)
        z = (sd * A1 + sdc * (A0 - A1) + A2) / jnp.maximum(nv, 1.0)
        out = jax.lax.dot_general(
            z, w2, (((1,), (0,)), ((), ())),
            preferred_element_type=jnp.float32) + b2
        out_ref[b : b + 1, :] = jnp.where(nv > 0, out, jnp.zeros_like(out))


def kernel(x, W1, b1, W2, b2):
    out_dim = W2.shape[1]
    return pl.pallas_call(
        _gnn_kernel,
        out_shape=jax.ShapeDtypeStruct((x.shape[0], out_dim), jnp.float32),
    )(x, W1, b1.reshape(1, -1), W2, b2.reshape(1, -1))


# speedup vs baseline: 1602.4505x; 1.0049x over previous
import jax
import jax.numpy as jnp
from jax.experimental import pallas as pl


_B, _N, _F = 8, 256, 6


def _dot(lhs, rhs, dims):
    return jax.lax.dot_general(lhs, rhs, (dims, ((), ())),
                               preferred_element_type=jnp.float32)


def _gnn_kernel(x_ref, w1_ref, b1_ref, w2_ref, b2_ref, out_ref):
    f = x_ref[...]
    bn = _B * _N
    absum = jnp.sum(jnp.abs(f), axis=1, keepdims=True)
    a = (absum != 0).astype(jnp.float32)
    c = ((f[:, 3:4] != 0) | (f[:, 4:5] != 0)).astype(jnp.float32)

    row_g = jax.lax.broadcasted_iota(jnp.int32, (bn, _B), 0) // _N
    col_b = jax.lax.broadcasted_iota(jnp.int32, (bn, _B), 1)
    G = (row_g == col_b).astype(jnp.float32)

    cnt = _dot(G, jnp.concatenate([a, a * c], axis=1), ((0,), (0,)))
    nv = cnt[:, 0:1]
    coefs = jnp.concatenate([cnt[:, 1:2], nv - cnt[:, 1:2] - 1.0], axis=1)
    bc = _dot(G, coefs, ((1,), (0,)))
    deg = a * (bc[:, 0:1] + c * bc[:, 1:2] + 1.0)
    d = jnp.where(deg > 0, 1.0 / jnp.sqrt(jnp.maximum(deg, 1e-12)), 0.0)
    dc = d * c
    dd1c = d * d * (1.0 - c)

    hw = _dot(f, w1_ref[...], ((1,), (0,)))

    Ld = G * d
    Ldc = G * dc
    S = _dot(jnp.concatenate([Ld, Ldc], axis=1), hw, ((0,), (0,)))
    S1 = S[_B:, :]
    SM = _dot(G, jnp.concatenate([S1, S[:_B, :] - S1], axis=1),
              ((1,), (0,)))
    hid = hw.shape[1]
    pre = d * SM[:, :hid] + dc * SM[:, hid:] + dd1c * hw + b1_ref[...]
    h1 = jnp.maximum(pre, 0.0)

    A = _dot(jnp.concatenate([Ld, Ldc, G * dd1c], axis=1), h1,
             ((0,), (0,)))
    sums = _dot(G, jnp.concatenate([d, dc], axis=1), ((0,), (0,)))
    A0, A1, A2 = A[:_B, :], A[_B:2 * _B, :], A[2 * _B:, :]
    z = (sums[:, 0:1] * A1 + sums[:, 1:2] * (A0 - A1) + A2) \
        / jnp.maximum(nv, 1.0)
    out = _dot(z, w2_ref[...], ((1,), (0,))) + b2_ref[...]
    out_ref[...] = jnp.where(nv > 0, out, jnp.zeros_like(out))


def kernel(x, W1, b1, W2, b2):
    b, n, f = x.shape
    return pl.pallas_call(
        _gnn_kernel,
        out_shape=jax.ShapeDtypeStruct((b, W2.shape[1]), jnp.float32),
    )(x.reshape(b * n, f), W1, b1.reshape(1, -1), W2, b2.reshape(1, -1))
